# Initial kernel scaffold; baseline (speedup 1.0000x reference)
import jax
import jax.numpy as jnp
from jax.experimental import pallas as pl


def kernel(x, router_w, w_in, w_out, bias):
    raise NotImplementedError("write your pallas kernel here")


# speedup vs baseline: 7.5210x; 7.5210x over previous
import functools

import jax
import jax.numpy as jnp
from jax.experimental import pallas as pl
from jax.experimental.pallas import tpu as pltpu

E = 64
K = 2
D = 1024
H = 2048

TM = 256
ALIGN = 8
NH = 2
HC = H // NH


def _gmm_body(poff_ref, xs_ref, wi_ref, wo_ref, out_ref):
    e = pl.program_id(0)
    hi = pl.program_id(1)

    @pl.when((e == 0) & (hi == 0))
    def _():
        out_ref[...] = jnp.zeros_like(out_ref)

    start = poff_ref[e]
    n = poff_ref[e + 1] - start
    nt = (n + TM - 1) // TM

    def body(i, carry):
        base = pl.multiple_of(start + i * TM, ALIGN)
        rows = xs_ref[pl.ds(base, TM), :]
        h = jax.lax.dot_general(
            rows, wi_ref[0], (((1,), (1,)), ((), ())),
            preferred_element_type=jnp.float32)
        o = jax.lax.dot_general(
            h, wo_ref[0], (((1,), (1,)), ((), ())),
            preferred_element_type=jnp.float32)
        rid = jax.lax.broadcasted_iota(jnp.int32, (TM, 1), 0) + i * TM
        o = jnp.where(rid < n, o, 0.0)
        out_ref[pl.ds(base, TM), :] += o
        return carry

    jax.lax.fori_loop(0, nt, body, 0)


@functools.partial(jax.jit, static_argnames=("ntot",))
def _grouped_matmul(xs, w_in, w_out, poff, ntot):
    return pl.pallas_call(
        _gmm_body,
        grid=(E, NH),
        in_specs=[
            pl.BlockSpec(memory_space=pltpu.SMEM),
            pl.BlockSpec((ntot, D), lambda e, hi: (0, 0)),
            pl.BlockSpec((1, HC, D), lambda e, hi: (e, hi, 0)),
            pl.BlockSpec((1, D, HC), lambda e, hi: (e, 0, hi)),
        ],
        out_specs=pl.BlockSpec((ntot, D), lambda e, hi: (0, 0)),
        out_shape=jax.ShapeDtypeStruct((ntot, D), jnp.float32),
        compiler_params=pltpu.CompilerParams(
            dimension_semantics=("arbitrary", "arbitrary"),
        ),
    )(poff, xs, w_in, w_out)


def kernel(x, router_w, w_in, w_out, bias):
    bsz, length, emb = x.shape
    t = bsz * length
    p = t * K
    npad = p + E * ALIGN
    ntot = npad + TM

    xf = x.reshape(t, emb)
    logits = xf @ router_w.T
    top_v, top_i = jax.lax.top_k(logits, K)
    gates = jax.nn.softmax(top_v, axis=1)

    fe = top_i.reshape(-1).astype(jnp.int32)
    onehot = (fe[:, None] == jnp.arange(E)[None, :]).astype(jnp.int32)
    counts = onehot.sum(axis=0)
    pc = ((counts + (ALIGN - 1)) // ALIGN) * ALIGN
    poff = jnp.concatenate(
        [jnp.zeros((1,), jnp.int32), jnp.cumsum(pc).astype(jnp.int32)])
    rank = jnp.cumsum(onehot, axis=0) - onehot
    rank_j = jnp.take_along_axis(rank, fe[:, None], 1)[:, 0]
    pos = poff[fe] + rank_j

    tok = (jnp.arange(p, dtype=jnp.int32) // K)
    src = jnp.zeros((ntot,), jnp.int32).at[pos].set(tok)
    gs = jnp.zeros((ntot, 1), jnp.float32).at[pos, 0].set(gates.reshape(-1))

    xs = xf[src] * gs
    out_sorted = _grouped_matmul(xs, w_in, w_out, poff, ntot)
    y = out_sorted[pos].reshape(t, K, emb).sum(axis=1) + bias
    return y.reshape(bsz, length, emb)
